# baseline (speedup 1.0000x reference)
import jax
import jax.numpy as jnp
from jax.experimental import pallas as pl
from jax.experimental.pallas import tpu as pltpu

NUM_GROUPS = 2
TOKENS_PER_GROUP = 4096
HIDDEN_DIM = 4096
NUM_EXPERTS = 64

BLOCK_T = 512


def _router_block(x_ref, w_ref, b_ref, probs_ref, logits_ref, zpart_ref):
    x = x_ref[...]
    w = w_ref[...]
    logits = jax.lax.dot_general(
        x, w, (((1,), (0,)), ((), ())), preferred_element_type=jnp.float32
    ) + b_ref[...]
    m = jnp.max(logits, axis=-1, keepdims=True)
    e = jnp.exp(logits - m)
    s = jnp.sum(e, axis=-1, keepdims=True)
    logits_ref[...] = logits
    probs_ref[...] = e / s
    log_z = m + jnp.log(s)
    partial = jnp.sum(log_z * log_z)
    zpart_ref[...] = jnp.full((8, 128), partial, jnp.float32)


def kernel(token_inputs, W, b, expert_capacity):
    del expert_capacity
    total_tokens = NUM_GROUPS * TOKENS_PER_GROUP
    n_blocks = total_tokens // BLOCK_T
    x = token_inputs.reshape(total_tokens, HIDDEN_DIM).astype(jnp.float32)
    b2 = b.reshape(1, NUM_EXPERTS).astype(jnp.float32)

    probs, logits, zparts = pl.pallas_call(
        _router_block,
        grid=(n_blocks,),
        in_specs=[
            pl.BlockSpec((BLOCK_T, HIDDEN_DIM), lambda i: (i, 0)),
            pl.BlockSpec((HIDDEN_DIM, NUM_EXPERTS), lambda i: (0, 0)),
            pl.BlockSpec((1, NUM_EXPERTS), lambda i: (0, 0)),
        ],
        out_specs=[
            pl.BlockSpec((BLOCK_T, NUM_EXPERTS), lambda i: (i, 0)),
            pl.BlockSpec((BLOCK_T, NUM_EXPERTS), lambda i: (i, 0)),
            pl.BlockSpec((8, 128), lambda i: (i, 0)),
        ],
        out_shape=[
            jax.ShapeDtypeStruct((total_tokens, NUM_EXPERTS), jnp.float32),
            jax.ShapeDtypeStruct((total_tokens, NUM_EXPERTS), jnp.float32),
            jax.ShapeDtypeStruct((n_blocks * 8, 128), jnp.float32),
        ],
        compiler_params=pltpu.CompilerParams(
            dimension_semantics=("parallel",),
        ),
    )(x, W.astype(jnp.float32).T, b2)

    router_probs = probs.reshape(NUM_GROUPS, TOKENS_PER_GROUP, NUM_EXPERTS)
    router_logits = logits.reshape(NUM_GROUPS, TOKENS_PER_GROUP, NUM_EXPERTS)
    router_z_loss = jnp.sum(zparts[::8, 0]) / total_tokens
    return (router_probs, router_logits, router_z_loss)


# speedup vs baseline: 1.1235x; 1.1235x over previous
import jax
import jax.numpy as jnp
from jax.experimental import pallas as pl

NUM_GROUPS = 2
TOKENS_PER_GROUP = 4096
HIDDEN_DIM = 4096
NUM_EXPERTS = 64

BLOCK_T = 512


def _router_block(x_ref, w_ref, b_ref, probs_ref, logits_ref, zacc_ref):
    g = pl.program_id(0)
    i = pl.program_id(1)
    x = x_ref[0]
    w = w_ref[...]
    logits = jax.lax.dot_general(
        x, w, (((1,), (1,)), ((), ())), preferred_element_type=jnp.float32
    ) + b_ref[...]
    m = jnp.max(logits, axis=-1, keepdims=True)
    e = jnp.exp(logits - m)
    s = jnp.sum(e, axis=-1, keepdims=True)
    logits_ref[0] = logits
    probs_ref[0] = e / s
    log_z = m + jnp.log(s)
    partial = jnp.sum(log_z * log_z).reshape(1, 1)

    @pl.when((g == 0) & (i == 0))
    def _init():
        zacc_ref[...] = jnp.zeros((1, 1), jnp.float32)

    zacc_ref[...] += partial

    last = (g == NUM_GROUPS - 1) & (i == pl.num_programs(1) - 1)

    @pl.when(last)
    def _finalize():
        zacc_ref[...] *= 1.0 / (NUM_GROUPS * TOKENS_PER_GROUP)


def kernel(token_inputs, W, b, expert_capacity):
    del expert_capacity
    n_blocks = TOKENS_PER_GROUP // BLOCK_T
    b2 = b.reshape(1, NUM_EXPERTS)

    probs, logits, zloss = pl.pallas_call(
        _router_block,
        grid=(NUM_GROUPS, n_blocks),
        in_specs=[
            pl.BlockSpec((1, BLOCK_T, HIDDEN_DIM), lambda g, i: (g, i, 0)),
            pl.BlockSpec((NUM_EXPERTS, HIDDEN_DIM), lambda g, i: (0, 0)),
            pl.BlockSpec((1, NUM_EXPERTS), lambda g, i: (0, 0)),
        ],
        out_specs=[
            pl.BlockSpec((1, BLOCK_T, NUM_EXPERTS), lambda g, i: (g, i, 0)),
            pl.BlockSpec((1, BLOCK_T, NUM_EXPERTS), lambda g, i: (g, i, 0)),
            pl.BlockSpec((1, 1), lambda g, i: (0, 0)),
        ],
        out_shape=[
            jax.ShapeDtypeStruct((NUM_GROUPS, TOKENS_PER_GROUP, NUM_EXPERTS), jnp.float32),
            jax.ShapeDtypeStruct((NUM_GROUPS, TOKENS_PER_GROUP, NUM_EXPERTS), jnp.float32),
            jax.ShapeDtypeStruct((1, 1), jnp.float32),
        ],
    )(token_inputs, W, b2)

    return (probs, logits, zloss.reshape(()))
